# Initial kernel scaffold; baseline (speedup 1.0000x reference)
import functools

import jax
import jax.numpy as jnp
from jax import lax
from jax.experimental import pallas as pl
from jax.experimental.pallas import tpu as pltpu
from jax.experimental.pallas import tpu_sc as plsc

N_NODES = 10000
N_EDGES = 320000
NP = 10240
NC = 2
NS = 16
NW = NC * NS
CH = 128
K = 80
E_PAD = NW * K * CH
ROWS_PER_S = NP // NS


def _mesh():
    return plsc.VectorSubcoreMesh(core_axis_name="c", subcore_axis_name="s")


def _aggregate(hw, srcr, dstr, d):
    @functools.partial(
        pl.kernel,
        mesh=_mesh(),
        out_type=jax.ShapeDtypeStruct((NC, NP, d), jnp.float32),
        scratch_types=[
            pltpu.VMEM((K, CH), jnp.int32),
            pltpu.VMEM((K, CH), jnp.int32),
            pltpu.VMEM((2, CH, d), jnp.float32),
            pltpu.VMEM((64, d), jnp.float32),
            pltpu.VMEM_SHARED((NP, d), jnp.float32),
            pltpu.SemaphoreType.DMA,
            pltpu.SemaphoreType.DMA,
        ],
    )
    def k(hw_hbm, src_hbm, dst_hbm, out_hbm, src_v, dst_v, rows_v, zb_v, acc_sh, g0, g1):
        c = lax.axis_index("c")
        s = lax.axis_index("s")
        w = s * NC + c

        @pl.loop(0, 64)
        def _(r):
            for t in range(d // 16):
                zb_v[r, pl.ds(t * 16, 16)] = jnp.zeros((16,), jnp.float32)

        @pl.loop(0, ROWS_PER_S // 64)
        def _(z):
            pltpu.sync_copy(zb_v, acc_sh.at[pl.ds(s * ROWS_PER_S + z * 64, 64)])

        plsc.subcore_barrier()

        pltpu.sync_copy(src_hbm.at[w], src_v)
        pltpu.sync_copy(dst_hbm.at[w], dst_v)

        @pl.loop(0, K, step=2)
        def _(j):
            h0 = pltpu.async_copy(hw_hbm.at[src_v.at[j]], rows_v.at[0], g0)
            h1 = pltpu.async_copy(hw_hbm.at[src_v.at[j + 1]], rows_v.at[1], g1)
            h0.wait()
            pltpu.sync_copy(rows_v.at[0], acc_sh.at[dst_v.at[j]], add=True)
            h1.wait()
            pltpu.sync_copy(rows_v.at[1], acc_sh.at[dst_v.at[j + 1]], add=True)

        plsc.subcore_barrier()

        pltpu.sync_copy(
            acc_sh.at[pl.ds(s * ROWS_PER_S, ROWS_PER_S)],
            out_hbm.at[c, pl.ds(s * ROWS_PER_S, ROWS_PER_S)],
        )

    return k(hw, srcr, dstr)


_DOT = functools.partial(
    lax.dot_general,
    dimension_numbers=(((1,), (0,)), ((), ())),
    preferred_element_type=jnp.float32,
    precision=lax.Precision.HIGHEST,
)

_R = 1024


def _matmul(x, w):
    din, dout = w.shape

    def body(x_ref, w_ref, o_ref):
        o_ref[...] = _DOT(x_ref[...], w_ref[...])

    return pl.pallas_call(
        body,
        grid=(NP // _R,),
        in_specs=[
            pl.BlockSpec((_R, din), lambda i: (i, 0)),
            pl.BlockSpec((din, dout), lambda i: (0, 0)),
        ],
        out_specs=pl.BlockSpec((_R, dout), lambda i: (i, 0)),
        out_shape=jax.ShapeDtypeStruct((NP, dout), jnp.float32),
    )(x, w)


def _relu_matmul(p, b, w):
    din, dout = w.shape

    def body(p_ref, b_ref, w_ref, o_ref):
        a = jnp.maximum(p_ref[0] + p_ref[1] + b_ref[...], 0.0)
        o_ref[...] = _DOT(a, w_ref[...])

    return pl.pallas_call(
        body,
        grid=(NP // _R,),
        in_specs=[
            pl.BlockSpec((NC, _R, din), lambda i: (0, i, 0)),
            pl.BlockSpec((1, din), lambda i: (0, 0)),
            pl.BlockSpec((din, dout), lambda i: (0, 0)),
        ],
        out_specs=pl.BlockSpec((_R, dout), lambda i: (i, 0)),
        out_shape=jax.ShapeDtypeStruct((NP, dout), jnp.float32),
    )(p, b, w)


def _bias_log_softmax(q, b, d):
    def body(q_ref, b_ref, o_ref):
        t = q_ref[0] + q_ref[1] + b_ref[...]
        m = jnp.max(t, axis=-1, keepdims=True)
        e = jnp.exp(t - m)
        ssum = jnp.sum(e, axis=-1, keepdims=True)
        o_ref[...] = t - m - jnp.log(ssum)

    return pl.pallas_call(
        body,
        grid=(NP // _R,),
        in_specs=[
            pl.BlockSpec((NC, _R, d), lambda i: (0, i, 0)),
            pl.BlockSpec((1, d), lambda i: (0, 0)),
        ],
        out_specs=pl.BlockSpec((_R, d), lambda i: (i, 0)),
        out_shape=jax.ShapeDtypeStruct((NP, d), jnp.float32),
    )(q, b)


def kernel(x, edge_index, W1, b1, W2, b2, W3, b3):
    src = edge_index[0].astype(jnp.int32)
    dst = edge_index[1].astype(jnp.int32)
    srcr = jnp.pad(src, (0, E_PAD - N_EDGES)).reshape(NW, K, CH)
    dstr = jnp.pad(dst, (0, E_PAD - N_EDGES), constant_values=NP - 1).reshape(NW, K, CH)

    xp = jnp.pad(x, ((0, NP - N_NODES), (0, 0)))
    b1r = b1.reshape(1, -1)
    b2r = b2.reshape(1, -1)
    b3r = b3.reshape(1, -1)

    h = _matmul(xp, W1)
    p = _aggregate(h, srcr, dstr, W1.shape[1])
    h = _relu_matmul(p, b1r, W2)
    p = _aggregate(h, srcr, dstr, W2.shape[1])
    h = _relu_matmul(p, b2r, W3)
    q = _aggregate(h, srcr, dstr, W3.shape[1])
    out = _bias_log_softmax(q, b3r, W3.shape[1])
    return out[:N_NODES]


# speedup vs baseline: 2.8866x; 2.8866x over previous
import functools

import jax
import jax.numpy as jnp
from jax import lax
from jax.experimental import pallas as pl
from jax.experimental.pallas import tpu as pltpu
from jax.experimental.pallas import tpu_sc as plsc

N_NODES = 10000
N_EDGES = 320000
NP = 10240
NC = 2
NS = 16
NW = NC * NS
CH = 128
K = 80
KH = K // 2
E_PAD = NW * K * CH
ROWS_PER_S = NP // NS


def _mesh():
    return plsc.VectorSubcoreMesh(core_axis_name="c", subcore_axis_name="s")


def _aggregate(hw, srcr, dstr, d):
    @functools.partial(
        pl.kernel,
        mesh=_mesh(),
        out_type=jax.ShapeDtypeStruct((NC, NP, d), jnp.float32),
        scratch_types=[
            pltpu.VMEM((KH, CH), jnp.int32),
            pltpu.VMEM((KH, CH), jnp.int32),
            pltpu.VMEM((2, CH, d), jnp.float32),
            pltpu.VMEM_SHARED((NP, d), jnp.float32),
            pltpu.SemaphoreType.DMA,
            pltpu.SemaphoreType.DMA,
        ],
    )
    def k(hw_hbm, src_hbm, dst_hbm, out_hbm, src_v, dst_v, rows_v, acc_sh, g0, g1):
        c = lax.axis_index("c")
        s = lax.axis_index("s")
        w = s * NC + c

        @pl.loop(0, CH)
        def _(r):
            for t in range(d // 16):
                rows_v[0, r, pl.ds(t * 16, 16)] = jnp.zeros((16,), jnp.float32)

        @pl.loop(0, ROWS_PER_S // CH)
        def _(z):
            pltpu.sync_copy(rows_v.at[0], acc_sh.at[pl.ds(s * ROWS_PER_S + z * CH, CH)])

        plsc.subcore_barrier()

        for half in range(2):
            pltpu.sync_copy(src_hbm.at[w, pl.ds(half * KH, KH)], src_v)
            pltpu.sync_copy(dst_hbm.at[w, pl.ds(half * KH, KH)], dst_v)

            @pl.loop(0, KH, step=2)
            def _(j):
                h0 = pltpu.async_copy(hw_hbm.at[src_v.at[j]], rows_v.at[0], g0)
                h1 = pltpu.async_copy(hw_hbm.at[src_v.at[j + 1]], rows_v.at[1], g1)
                h0.wait()
                pltpu.sync_copy(rows_v.at[0], acc_sh.at[dst_v.at[j]], add=True)
                h1.wait()
                pltpu.sync_copy(rows_v.at[1], acc_sh.at[dst_v.at[j + 1]], add=True)

        plsc.subcore_barrier()

        pltpu.sync_copy(
            acc_sh.at[pl.ds(s * ROWS_PER_S, ROWS_PER_S)],
            out_hbm.at[c, pl.ds(s * ROWS_PER_S, ROWS_PER_S)],
        )

    return k(hw, srcr, dstr)


_DOT = functools.partial(
    lax.dot_general,
    dimension_numbers=(((1,), (0,)), ((), ())),
    preferred_element_type=jnp.float32,
    precision=lax.Precision.HIGHEST,
)

_R = 1024


def _matmul(x, w):
    din, dout = w.shape

    def body(x_ref, w_ref, o_ref):
        o_ref[...] = _DOT(x_ref[...], w_ref[...])

    return pl.pallas_call(
        body,
        grid=(NP // _R,),
        in_specs=[
            pl.BlockSpec((_R, din), lambda i: (i, 0)),
            pl.BlockSpec((din, dout), lambda i: (0, 0)),
        ],
        out_specs=pl.BlockSpec((_R, dout), lambda i: (i, 0)),
        out_shape=jax.ShapeDtypeStruct((NP, dout), jnp.float32),
    )(x, w)


def _relu_matmul(p, b, w):
    din, dout = w.shape

    def body(p_ref, b_ref, w_ref, o_ref):
        a = jnp.maximum(p_ref[0] + p_ref[1] + b_ref[...], 0.0)
        o_ref[...] = _DOT(a, w_ref[...])

    return pl.pallas_call(
        body,
        grid=(NP // _R,),
        in_specs=[
            pl.BlockSpec((NC, _R, din), lambda i: (0, i, 0)),
            pl.BlockSpec((1, din), lambda i: (0, 0)),
            pl.BlockSpec((din, dout), lambda i: (0, 0)),
        ],
        out_specs=pl.BlockSpec((_R, dout), lambda i: (i, 0)),
        out_shape=jax.ShapeDtypeStruct((NP, dout), jnp.float32),
    )(p, b, w)


def _relu_bias(p, b):
    d = p.shape[-1]

    def body(p_ref, b_ref, o_ref):
        o_ref[...] = jnp.maximum(p_ref[0] + p_ref[1] + b_ref[...], 0.0)

    return pl.pallas_call(
        body,
        grid=(NP // _R,),
        in_specs=[
            pl.BlockSpec((NC, _R, d), lambda i: (0, i, 0)),
            pl.BlockSpec((1, d), lambda i: (0, 0)),
        ],
        out_specs=pl.BlockSpec((_R, d), lambda i: (i, 0)),
        out_shape=jax.ShapeDtypeStruct((NP, d), jnp.float32),
    )(p, b)


def _matmul_bias_log_softmax(q, w, b):
    din, dout = w.shape

    def body(q_ref, w_ref, b_ref, o_ref):
        t = _DOT(q_ref[0] + q_ref[1], w_ref[...]) + b_ref[...]
        m = jnp.max(t, axis=-1, keepdims=True)
        e = jnp.exp(t - m)
        ssum = jnp.sum(e, axis=-1, keepdims=True)
        o_ref[...] = t - m - jnp.log(ssum)

    return pl.pallas_call(
        body,
        grid=(NP // _R,),
        in_specs=[
            pl.BlockSpec((NC, _R, din), lambda i: (0, i, 0)),
            pl.BlockSpec((din, dout), lambda i: (0, 0)),
            pl.BlockSpec((1, dout), lambda i: (0, 0)),
        ],
        out_specs=pl.BlockSpec((_R, dout), lambda i: (i, 0)),
        out_shape=jax.ShapeDtypeStruct((NP, dout), jnp.float32),
    )(q, w, b)


def kernel(x, edge_index, W1, b1, W2, b2, W3, b3):
    src = edge_index[0].astype(jnp.int32)
    dst = edge_index[1].astype(jnp.int32)
    srcr = jnp.pad(src, (0, E_PAD - N_EDGES)).reshape(NW, K, CH)
    dstr = jnp.pad(dst, (0, E_PAD - N_EDGES), constant_values=NP - 1).reshape(NW, K, CH)

    xp = jnp.pad(x, ((0, NP - N_NODES), (0, 0)))
    b1r = b1.reshape(1, -1)
    b2r = b2.reshape(1, -1)
    b3r = b3.reshape(1, -1)

    h = _matmul(xp, W1)
    p = _aggregate(h, srcr, dstr, W1.shape[1])
    h = _relu_matmul(p, b1r, W2)
    p = _aggregate(h, srcr, dstr, W2.shape[1])
    h = _relu_bias(p, b2r)
    q = _aggregate(h, srcr, dstr, W2.shape[1])
    out = _matmul_bias_log_softmax(q, W3, b3r)
    return out[:N_NODES]


# speedup vs baseline: 3.0969x; 1.0729x over previous
import functools

import jax
import jax.numpy as jnp
from jax import lax
from jax.experimental import pallas as pl
from jax.experimental.pallas import tpu as pltpu
from jax.experimental.pallas import tpu_sc as plsc

N_NODES = 10000
N_EDGES = 320000
NP = 10240
NC = 2
NS = 16
NW = NC * NS
CH = 80
K = 128
NQ = 4
CPQ = K // NQ
NB = 4
E_PAD = NW * K * CH
ROWS_PER_S = NP // NS


def _mesh():
    return plsc.VectorSubcoreMesh(core_axis_name="c", subcore_axis_name="s")


def _aggregate(hw, srcr, dstr, d):
    @functools.partial(
        pl.kernel,
        mesh=_mesh(),
        out_type=jax.ShapeDtypeStruct((NC, NP, d), jnp.float32),
        scratch_types=[
            pltpu.VMEM((CPQ, CH), jnp.int32),
            pltpu.VMEM((CPQ, CH), jnp.int32),
            pltpu.VMEM((NB, CH, d), jnp.float32),
            pltpu.VMEM_SHARED((NP, d), jnp.float32),
        ] + [pltpu.SemaphoreType.DMA] * NB,
    )
    def k(hw_hbm, src_hbm, dst_hbm, out_hbm, src_v, dst_v, rows_v, acc_sh, *gsems):
        c = lax.axis_index("c")
        s = lax.axis_index("s")
        w = s * NC + c

        @pl.loop(0, CH)
        def _(r):
            for t in range(d // 16):
                rows_v[0, r, pl.ds(t * 16, 16)] = jnp.zeros((16,), jnp.float32)

        @pl.loop(0, ROWS_PER_S // CH)
        def _(z):
            pltpu.sync_copy(rows_v.at[0], acc_sh.at[pl.ds(s * ROWS_PER_S + z * CH, CH)])

        plsc.subcore_barrier()

        def fire_gather(jj, b):
            pltpu.async_copy(hw_hbm.at[src_v.at[jj]], rows_v.at[b], gsems[b])

        def wait_gather(jj, b):
            pltpu.make_async_copy(hw_hbm.at[src_v.at[jj]], rows_v.at[b], gsems[b]).wait()

        def scatter_add(jj, b):
            pltpu.sync_copy(rows_v.at[b], acc_sh.at[dst_v.at[jj]], add=True)

        @pl.loop(0, NQ)
        def _(q):
            pltpu.sync_copy(src_hbm.at[w, pl.ds(q * CPQ, CPQ)], src_v)
            pltpu.sync_copy(dst_hbm.at[w, pl.ds(q * CPQ, CPQ)], dst_v)
            for i in range(NB):
                fire_gather(i, i)

            @pl.loop(0, CPQ - NB, step=NB)
            def _(j):
                for i in range(NB):
                    jj = j + i
                    wait_gather(jj, i)
                    scatter_add(jj, i)
                    fire_gather(jj + NB, i)

            for i in range(NB):
                jj = CPQ - NB + i
                wait_gather(jj, i)
                scatter_add(jj, i)

        plsc.subcore_barrier()

        pltpu.sync_copy(
            acc_sh.at[pl.ds(s * ROWS_PER_S, ROWS_PER_S)],
            out_hbm.at[c, pl.ds(s * ROWS_PER_S, ROWS_PER_S)],
        )

    return k(hw, srcr, dstr)


_DOT = functools.partial(
    lax.dot_general,
    dimension_numbers=(((1,), (0,)), ((), ())),
    preferred_element_type=jnp.float32,
    precision=lax.Precision.HIGHEST,
)

_R = 1024


def _matmul(x, w):
    din, dout = w.shape

    def body(x_ref, w_ref, o_ref):
        o_ref[...] = _DOT(x_ref[...], w_ref[...])

    return pl.pallas_call(
        body,
        grid=(NP // _R,),
        in_specs=[
            pl.BlockSpec((_R, din), lambda i: (i, 0)),
            pl.BlockSpec((din, dout), lambda i: (0, 0)),
        ],
        out_specs=pl.BlockSpec((_R, dout), lambda i: (i, 0)),
        out_shape=jax.ShapeDtypeStruct((NP, dout), jnp.float32),
    )(x, w)


def _relu_matmul(p, b, w):
    din, dout = w.shape

    def body(p_ref, b_ref, w_ref, o_ref):
        a = jnp.maximum(p_ref[0] + p_ref[1] + b_ref[...], 0.0)
        o_ref[...] = _DOT(a, w_ref[...])

    return pl.pallas_call(
        body,
        grid=(NP // _R,),
        in_specs=[
            pl.BlockSpec((NC, _R, din), lambda i: (0, i, 0)),
            pl.BlockSpec((1, din), lambda i: (0, 0)),
            pl.BlockSpec((din, dout), lambda i: (0, 0)),
        ],
        out_specs=pl.BlockSpec((_R, dout), lambda i: (i, 0)),
        out_shape=jax.ShapeDtypeStruct((NP, dout), jnp.float32),
    )(p, b, w)


def _relu_bias(p, b):
    d = p.shape[-1]

    def body(p_ref, b_ref, o_ref):
        o_ref[...] = jnp.maximum(p_ref[0] + p_ref[1] + b_ref[...], 0.0)

    return pl.pallas_call(
        body,
        grid=(NP // _R,),
        in_specs=[
            pl.BlockSpec((NC, _R, d), lambda i: (0, i, 0)),
            pl.BlockSpec((1, d), lambda i: (0, 0)),
        ],
        out_specs=pl.BlockSpec((_R, d), lambda i: (i, 0)),
        out_shape=jax.ShapeDtypeStruct((NP, d), jnp.float32),
    )(p, b)


def _matmul_bias_log_softmax(q, w, b):
    din, dout = w.shape

    def body(q_ref, w_ref, b_ref, o_ref):
        t = _DOT(q_ref[0] + q_ref[1], w_ref[...]) + b_ref[...]
        m = jnp.max(t, axis=-1, keepdims=True)
        e = jnp.exp(t - m)
        ssum = jnp.sum(e, axis=-1, keepdims=True)
        o_ref[...] = t - m - jnp.log(ssum)

    return pl.pallas_call(
        body,
        grid=(NP // _R,),
        in_specs=[
            pl.BlockSpec((NC, _R, din), lambda i: (0, i, 0)),
            pl.BlockSpec((din, dout), lambda i: (0, 0)),
            pl.BlockSpec((1, dout), lambda i: (0, 0)),
        ],
        out_specs=pl.BlockSpec((_R, dout), lambda i: (i, 0)),
        out_shape=jax.ShapeDtypeStruct((NP, dout), jnp.float32),
    )(q, w, b)


def kernel(x, edge_index, W1, b1, W2, b2, W3, b3):
    src = edge_index[0].astype(jnp.int32)
    dst = edge_index[1].astype(jnp.int32)
    srcr = jnp.pad(src, (0, E_PAD - N_EDGES)).reshape(NW, K, CH)
    dstr = jnp.pad(dst, (0, E_PAD - N_EDGES), constant_values=NP - 1).reshape(NW, K, CH)

    xp = jnp.pad(x, ((0, NP - N_NODES), (0, 0)))
    b1r = b1.reshape(1, -1)
    b2r = b2.reshape(1, -1)
    b3r = b3.reshape(1, -1)

    h = _matmul(xp, W1)
    p = _aggregate(h, srcr, dstr, W1.shape[1])
    h = _relu_matmul(p, b1r, W2)
    p = _aggregate(h, srcr, dstr, W2.shape[1])
    h = _relu_bias(p, b2r)
    q = _aggregate(h, srcr, dstr, W2.shape[1])
    out = _matmul_bias_log_softmax(q, W3, b3r)
    return out[:N_NODES]


# speedup vs baseline: 7.8464x; 2.5336x over previous
import functools

import jax
import jax.numpy as jnp
from jax import lax
from jax.experimental import pallas as pl
from jax.experimental.pallas import tpu as pltpu
from jax.experimental.pallas import tpu_sc as plsc

N_NODES = 10000
N_EDGES = 320000
NP = 10240
NC = 2
NS = 16
NW = NC * NS
CH = 80
K = 128
NB = 4
HD = 64
E_PAD = NW * K * CH
ROWS_PER_S = NP // NS


def _aggregate(hw2, srcr, dstr):
    mesh = plsc.VectorSubcoreMesh(core_axis_name="c", subcore_axis_name="s")

    @functools.partial(
        pl.kernel,
        mesh=mesh,
        compiler_params=pltpu.CompilerParams(use_tc_tiling_on_sc=False),
        out_type=jax.ShapeDtypeStruct((NC, 2, NP, HD), jnp.float32),
        scratch_types=[
            pltpu.VMEM((K, CH), jnp.int32),
            pltpu.VMEM((K, CH), jnp.int32),
            pltpu.VMEM((NB, CH, HD), jnp.float32),
            pltpu.VMEM_SHARED((NP, HD), jnp.float32),
            pltpu.VMEM_SHARED((NP, HD), jnp.float32),
        ] + [pltpu.SemaphoreType.DMA] * NB,
    )
    def k(hw_hbm, src_hbm, dst_hbm, out_hbm, src_v, dst_v, rows_v, tab_sh, acc_sh, *gsems):
        c = lax.axis_index("c")
        s = lax.axis_index("s")
        w = s * NC + c

        pltpu.sync_copy(src_hbm.at[w], src_v)
        pltpu.sync_copy(dst_hbm.at[w], dst_v)

        def fire_gather(jj, b):
            pltpu.async_copy(tab_sh.at[src_v.at[jj]], rows_v.at[b], gsems[b])

        def wait_gather(jj, b):
            pltpu.make_async_copy(tab_sh.at[src_v.at[jj]], rows_v.at[b], gsems[b]).wait()

        def scatter_add(jj, b):
            pltpu.sync_copy(rows_v.at[b], acc_sh.at[dst_v.at[jj]], add=True)

        for h in range(2):
            @pl.loop(0, CH)
            def _(r):
                for t in range(HD // 16):
                    rows_v[0, r, pl.ds(t * 16, 16)] = jnp.zeros((16,), jnp.float32)

            pltpu.sync_copy(
                hw_hbm.at[h, pl.ds(s * ROWS_PER_S, ROWS_PER_S)],
                tab_sh.at[pl.ds(s * ROWS_PER_S, ROWS_PER_S)],
            )

            @pl.loop(0, ROWS_PER_S // CH)
            def _(z):
                pltpu.sync_copy(rows_v.at[0], acc_sh.at[pl.ds(s * ROWS_PER_S + z * CH, CH)])

            plsc.subcore_barrier()

            for i in range(NB):
                fire_gather(i, i)

            @pl.loop(0, K - NB, step=NB)
            def _(j):
                for i in range(NB):
                    jj = j + i
                    wait_gather(jj, i)
                    scatter_add(jj, i)
                    fire_gather(jj + NB, i)

            for i in range(NB):
                jj = K - NB + i
                wait_gather(jj, i)
                scatter_add(jj, i)

            plsc.subcore_barrier()

            pltpu.sync_copy(
                acc_sh.at[pl.ds(s * ROWS_PER_S, ROWS_PER_S)],
                out_hbm.at[c, h, pl.ds(s * ROWS_PER_S, ROWS_PER_S)],
            )

    return k(hw2, srcr, dstr)


_DOT = functools.partial(
    lax.dot_general,
    dimension_numbers=(((1,), (0,)), ((), ())),
    preferred_element_type=jnp.float32,
    precision=lax.Precision.HIGHEST,
)

_R = 1024


def _matmul_split(x, w):
    def body(x_ref, w_ref, o_ref):
        xx = x_ref[...]
        o_ref[0] = _DOT(xx, w_ref[:, :HD])
        o_ref[1] = _DOT(xx, w_ref[:, HD:])

    return pl.pallas_call(
        body,
        grid=(NP // _R,),
        in_specs=[
            pl.BlockSpec((_R, 128), lambda i: (i, 0)),
            pl.BlockSpec((128, 128), lambda i: (0, 0)),
        ],
        out_specs=pl.BlockSpec((2, _R, HD), lambda i: (0, i, 0)),
        out_shape=jax.ShapeDtypeStruct((2, NP, HD), jnp.float32),
    )(x, w)


def _relu_matmul_split(p, b, w):
    def body(p_ref, b_ref, w_ref, o_ref):
        a = jnp.concatenate(
            [
                jnp.maximum(p_ref[0, 0] + p_ref[1, 0] + b_ref[:, :HD], 0.0),
                jnp.maximum(p_ref[0, 1] + p_ref[1, 1] + b_ref[:, HD:], 0.0),
            ],
            axis=-1,
        )
        o_ref[0] = _DOT(a, w_ref[:, :HD])
        o_ref[1] = _DOT(a, w_ref[:, HD:])

    return pl.pallas_call(
        body,
        grid=(NP // _R,),
        in_specs=[
            pl.BlockSpec((NC, 2, _R, HD), lambda i: (0, 0, i, 0)),
            pl.BlockSpec((1, 128), lambda i: (0, 0)),
            pl.BlockSpec((128, 128), lambda i: (0, 0)),
        ],
        out_specs=pl.BlockSpec((2, _R, HD), lambda i: (0, i, 0)),
        out_shape=jax.ShapeDtypeStruct((2, NP, HD), jnp.float32),
    )(p, b, w)


def _relu_split(p, b):
    def body(p_ref, b_ref, o_ref):
        o_ref[0] = jnp.maximum(p_ref[0, 0] + p_ref[1, 0] + b_ref[:, :HD], 0.0)
        o_ref[1] = jnp.maximum(p_ref[0, 1] + p_ref[1, 1] + b_ref[:, HD:], 0.0)

    return pl.pallas_call(
        body,
        grid=(NP // _R,),
        in_specs=[
            pl.BlockSpec((NC, 2, _R, HD), lambda i: (0, 0, i, 0)),
            pl.BlockSpec((1, 128), lambda i: (0, 0)),
        ],
        out_specs=pl.BlockSpec((2, _R, HD), lambda i: (0, i, 0)),
        out_shape=jax.ShapeDtypeStruct((2, NP, HD), jnp.float32),
    )(p, b)


def _matmul_bias_log_softmax(q, w, b):
    dout = w.shape[1]

    def body(q_ref, w_ref, b_ref, o_ref):
        a = jnp.concatenate(
            [q_ref[0, 0] + q_ref[1, 0], q_ref[0, 1] + q_ref[1, 1]], axis=-1
        )
        t = _DOT(a, w_ref[...]) + b_ref[...]
        m = jnp.max(t, axis=-1, keepdims=True)
        e = jnp.exp(t - m)
        ssum = jnp.sum(e, axis=-1, keepdims=True)
        o_ref[...] = t - m - jnp.log(ssum)

    return pl.pallas_call(
        body,
        grid=(NP // _R,),
        in_specs=[
            pl.BlockSpec((NC, 2, _R, HD), lambda i: (0, 0, i, 0)),
            pl.BlockSpec((128, dout), lambda i: (0, 0)),
            pl.BlockSpec((1, dout), lambda i: (0, 0)),
        ],
        out_specs=pl.BlockSpec((_R, dout), lambda i: (i, 0)),
        out_shape=jax.ShapeDtypeStruct((NP, dout), jnp.float32),
    )(q, w, b)


def kernel(x, edge_index, W1, b1, W2, b2, W3, b3):
    src = edge_index[0].astype(jnp.int32)
    dst = edge_index[1].astype(jnp.int32)
    srcr = jnp.pad(src, (0, E_PAD - N_EDGES)).reshape(NW, K, CH)
    dstr = jnp.pad(dst, (0, E_PAD - N_EDGES), constant_values=NP - 1).reshape(NW, K, CH)

    xp = jnp.pad(x, ((0, NP - N_NODES), (0, 0)))
    b1r = b1.reshape(1, -1)
    b2r = b2.reshape(1, -1)
    b3r = b3.reshape(1, -1)

    h = _matmul_split(xp, W1)
    p = _aggregate(h, srcr, dstr)
    h = _relu_matmul_split(p, b1r, W2)
    p = _aggregate(h, srcr, dstr)
    h = _relu_split(p, b2r)
    q = _aggregate(h, srcr, dstr)
    out = _matmul_bias_log_softmax(q, W3, b3r)
    return out[:N_NODES]
